# baseline (speedup 1.0000x reference)
import functools

import jax
import jax.numpy as jnp
from jax import lax
from jax.experimental import pallas as pl
from jax.experimental.pallas import tpu as pltpu
from jax.experimental.pallas import tpu_sc as plsc

NC = 2
NS = 16
CD = 48
CB = 80

_MESH = plsc.VectorSubcoreMesh(
    core_axis_name="c", subcore_axis_name="s", num_cores=NC, num_subcores=NS
)


def _round_up(a, b):
    return (a + b - 1) // b * b


def _pad_edges(v, total, base, spread):
    if v.shape[0] == total:
        return v
    pad = base + jnp.arange(total - v.shape[0], dtype=v.dtype) % spread
    return jnp.concatenate([v, pad])


def _make_deg_kernel(npad, tpr, rpt):
    zb = 160

    @functools.partial(
        pl.kernel,
        out_type=jax.ShapeDtypeStruct((NC * npad,), jnp.float32),
        mesh=_MESH,
        scratch_types=[
            pltpu.VMEM((rpt, CD), jnp.int32),
            pltpu.VMEM((CD,), jnp.float32),
            pltpu.VMEM((zb,), jnp.float32),
            pltpu.VMEM_SHARED((npad,), jnp.float32),
            pltpu.SemaphoreType.DMA,
        ],
    )
    def deg_kernel(dst_hbm, out_hbm, didx, ones_v, zbuf, acc, sem):
        c = lax.axis_index("c")
        t = lax.axis_index("s")
        w = c * NS + t

        @pl.loop(0, zb, step=16)
        def _(j):
            zbuf[pl.ds(j, 16)] = jnp.zeros((16,), jnp.float32)

        @pl.loop(0, tpr, step=zb)
        def _(q):
            pltpu.sync_copy(zbuf, acc.at[pl.ds(t * tpr + q, zb)])

        @pl.loop(0, CD, step=16)
        def _(j):
            ones_v[pl.ds(j, 16)] = jnp.full((16,), 1.0, jnp.float32)

        pltpu.sync_copy(dst_hbm.at[pl.ds(w * rpt, rpt)], didx)
        plsc.subcore_barrier()

        @pl.loop(0, rpt)
        def _(k):
            pltpu.sync_copy(ones_v, acc.at[didx.at[k]], add=True)

        plsc.subcore_barrier()
        pltpu.sync_copy(
            acc.at[pl.ds(t * tpr, tpr)],
            out_hbm.at[pl.ds(c * npad + t * tpr, tpr)],
        )

    return deg_kernel


SCK = 16
NRING = 3


def _make_agg_kernel(nsh, npad, tpr, rpt):
    spc = nsh // NC
    nsup = rpt // SCK

    @functools.partial(
        pl.kernel,
        out_type=jax.ShapeDtypeStruct((nsh, npad, 128), jnp.float32),
        mesh=_MESH,
        scratch_types=[
            pltpu.VMEM((SCK, CB), jnp.int32),
            pltpu.VMEM((SCK, CB), jnp.int32),
            [pltpu.VMEM((CB, 128), jnp.float32)] * NRING,
            pltpu.VMEM_SHARED((npad, 128), jnp.float32),
            [pltpu.SemaphoreType.DMA] * NRING,
            [pltpu.SemaphoreType.DMA] * NRING,
        ],
    )
    def agg_kernel(y_hbm, src_hbm, dst_hbm, out_hbm,
                   srcb, dstb, rbufs, acc, gsems, ssems):
        c = lax.axis_index("c")
        t = lax.axis_index("s")

        for p in range(spc):
            sh = c * spc + p

            @pl.loop(0, CB)
            def _(r):
                @pl.loop(0, 128, step=16)
                def _(j):
                    rbufs[0][r, pl.ds(j, 16)] = jnp.zeros((16,), jnp.float32)

            @pl.loop(0, tpr, step=CB)
            def _(q):
                pltpu.sync_copy(rbufs[0], acc.at[pl.ds(t * tpr + q, CB)])

            plsc.subcore_barrier()

            @pl.loop(0, nsup)
            def _(u):
                base = t * rpt + u * SCK
                pltpu.sync_copy(src_hbm.at[pl.ds(base, SCK)], srcb)
                pltpu.sync_copy(dst_hbm.at[pl.ds(base, SCK)], dstb)
                gds = [None] * SCK
                sds = [None] * SCK
                ytab = y_hbm.at[sh]
                gds[0] = pltpu.async_copy(ytab.at[srcb.at[0]], rbufs[0],
                                          gsems[0])
                for k in range(SCK):
                    b = k % NRING
                    if k + 1 < SCK:
                        nb = (k + 1) % NRING
                        if k + 1 >= NRING:
                            sds[k + 1 - NRING].wait()
                        gds[k + 1] = pltpu.async_copy(
                            ytab.at[srcb.at[k + 1]], rbufs[nb], gsems[nb]
                        )
                    gds[k].wait()
                    sds[k] = pltpu.async_copy(
                        rbufs[b], acc.at[dstb.at[k]], ssems[b], add=True
                    )
                for k in range(SCK - NRING, SCK):
                    sds[k].wait()

            plsc.subcore_barrier()
            pltpu.sync_copy(
                acc.at[pl.ds(t * tpr, tpr)], out_hbm.at[sh, pl.ds(t * tpr, tpr)]
            )

    return agg_kernel


def _dinv_of(dcol_block):
    deg = dcol_block[:, 0:1] + dcol_block[:, 1:2] + 1.0
    return lax.rsqrt(deg)


def _tc_scale(deg_col, x, bm):
    n, d = x.shape

    def body(dcol_ref, x_ref, y_ref):
        y = x_ref[...] * _dinv_of(dcol_ref[...])
        y_ref[0] = y[:, 0:128]
        y_ref[1] = y[:, 128:256]

    return pl.pallas_call(
        body,
        grid=(n // bm,),
        in_specs=[
            pl.BlockSpec((bm, 2), lambda i: (i, 0)),
            pl.BlockSpec((bm, d), lambda i: (i, 0)),
        ],
        out_specs=pl.BlockSpec((2, bm, 128), lambda i: (0, i, 0)),
        out_shape=jax.ShapeDtypeStruct((2, n, 128), jnp.float32),
    )(deg_col, x)


def _tc_layer1(deg_col, y, agg1, W1, b1, bm):
    n = y.shape[1]
    d = 2 * y.shape[2]
    h0 = W1.shape[1]

    def body(dcol_ref, y_ref, a_ref, w_ref, b_ref, o_ref):
        dinv = _dinv_of(dcol_ref[...])
        agg = jnp.concatenate([a_ref[0], a_ref[1]], axis=1)
        yb = jnp.concatenate([y_ref[0], y_ref[1]], axis=1)
        xa = (agg + yb) * dinv
        h = jnp.dot(xa.astype(jnp.bfloat16), w_ref[...],
                    preferred_element_type=jnp.float32)
        h = jnp.maximum(h + b_ref[...], 0.0) * dinv
        for s in range(4):
            o_ref[s] = h[:, 128 * s:128 * (s + 1)]

    return pl.pallas_call(
        body,
        grid=(n // bm,),
        in_specs=[
            pl.BlockSpec((bm, 2), lambda i: (i, 0)),
            pl.BlockSpec((2, bm, 128), lambda i: (0, i, 0)),
            pl.BlockSpec((2, bm, 128), lambda i: (0, i, 0)),
            pl.BlockSpec((d, h0), lambda i: (0, 0)),
            pl.BlockSpec((1, h0), lambda i: (0, 0)),
        ],
        out_specs=pl.BlockSpec((4, bm, 128), lambda i: (0, i, 0)),
        out_shape=jax.ShapeDtypeStruct((4, n, 128), jnp.float32),
    )(deg_col, y, agg1, W1, b1)


def _tc_layer2(deg_col, hs, agg2, Wmu, bmu, Wls, bls, bm):
    n = hs.shape[1]
    h0 = 4 * hs.shape[2]
    h1 = Wmu.shape[1]

    def body(dcol_ref, h_ref, a_ref, wm_ref, bm_ref, wl_ref, bl_ref, o_ref):
        dinv = _dinv_of(dcol_ref[...])
        agg = jnp.concatenate(
            [a_ref[0], a_ref[1], a_ref[2], a_ref[3]], axis=1
        )
        hb = jnp.concatenate(
            [h_ref[0], h_ref[1], h_ref[2], h_ref[3]], axis=1
        )
        ha = ((agg + hb) * dinv).astype(jnp.bfloat16)
        mu = jnp.dot(ha, wm_ref[...], preferred_element_type=jnp.float32)
        mu = mu + bm_ref[...]
        ls = jnp.dot(ha, wl_ref[...], preferred_element_type=jnp.float32)
        ls = ls + bl_ref[...]
        sg = jnp.where(ls > 0.0, ls, jnp.exp(jnp.minimum(ls, 0.0)) - 1.0)
        o_ref[0] = mu
        o_ref[1] = sg + (1.0 + 1e-14)

    return pl.pallas_call(
        body,
        grid=(n // bm,),
        in_specs=[
            pl.BlockSpec((bm, 2), lambda i: (i, 0)),
            pl.BlockSpec((4, bm, 128), lambda i: (0, i, 0)),
            pl.BlockSpec((4, bm, 128), lambda i: (0, i, 0)),
            pl.BlockSpec((h0, h1), lambda i: (0, 0)),
            pl.BlockSpec((1, h1), lambda i: (0, 0)),
            pl.BlockSpec((h0, h1), lambda i: (0, 0)),
            pl.BlockSpec((1, h1), lambda i: (0, 0)),
        ],
        out_specs=pl.BlockSpec((2, bm, h1), lambda i: (0, i, 0)),
        out_shape=jax.ShapeDtypeStruct((2, n, h1), jnp.float32),
    )(deg_col, hs, agg2, Wmu, bmu, Wls, bls)


def kernel(x, edge_index, W1, b1, Wmu, bmu, Wls, bls):
    n, d_in = x.shape
    h0 = W1.shape[1]
    h1 = Wmu.shape[1]
    e = edge_index.shape[1]
    assert d_in % 128 == 0 and h0 % 128 == 0

    tpr = _round_up(-(-n // NS), 160)
    npad = tpr * NS
    sentinel = npad - 1

    src = edge_index[0]
    dst = edge_index[1]

    ed = _round_up(e, NC * NS * CD * 8)
    dst_deg = _pad_edges(dst, ed, n, npad - n).reshape(ed // CD, CD)
    rpt_deg = ed // CD // (NC * NS)

    ea = _round_up(e, NS * CB * 8)
    src_a = _pad_edges(src, ea, 0, n).reshape(ea // CB, CB)
    dst_a = _pad_edges(dst, ea, n, npad - n).reshape(ea // CB, CB)
    rpt_a = ea // CB // NS

    deg_parts = _make_deg_kernel(npad, tpr, rpt_deg)(dst_deg)
    deg_col = jnp.transpose(deg_parts.reshape(NC, npad))

    bm = 1000 if n % 1000 == 0 else n
    y = _tc_scale(deg_col, x, bm)

    agg1 = _make_agg_kernel(2, npad, tpr, rpt_a)(y, src_a, dst_a)
    hs = _tc_layer1(deg_col, y, agg1, W1.astype(jnp.bfloat16),
                    b1.reshape(1, h0), bm)

    agg2 = _make_agg_kernel(4, npad, tpr, rpt_a)(hs, src_a, dst_a)
    return _tc_layer2(
        deg_col, hs, agg2, Wmu.astype(jnp.bfloat16), bmu.reshape(1, h1),
        Wls.astype(jnp.bfloat16), bls.reshape(1, h1), bm
    )


# speedup vs baseline: 1.0739x; 1.0739x over previous
import functools

import jax
import jax.numpy as jnp
from jax import lax
from jax.experimental import pallas as pl
from jax.experimental.pallas import tpu as pltpu
from jax.experimental.pallas import tpu_sc as plsc

NC = 2
NS = 16
CB = 80

_MESH = plsc.VectorSubcoreMesh(
    core_axis_name="c", subcore_axis_name="s", num_cores=NC, num_subcores=NS
)


def _round_up(a, b):
    return (a + b - 1) // b * b


def _pad_edges(v, total, base, spread):
    if v.shape[0] == total:
        return v
    pad = base + jnp.arange(total - v.shape[0], dtype=v.dtype) % spread
    return jnp.concatenate([v, pad])


def _make_deg_kernel(npad, tpr, rpt):
    zb = 160

    @functools.partial(
        pl.kernel,
        out_type=jax.ShapeDtypeStruct((NC * npad,), jnp.float32),
        mesh=_MESH,
        scratch_types=[
            pltpu.VMEM((rpt, CB), jnp.int32),
            pltpu.VMEM((CB,), jnp.float32),
            pltpu.VMEM((zb,), jnp.float32),
            pltpu.VMEM_SHARED((npad,), jnp.float32),
            pltpu.SemaphoreType.DMA,
        ],
    )
    def deg_kernel(dst_hbm, out_hbm, didx, ones_v, zbuf, acc, sem):
        c = lax.axis_index("c")
        t = lax.axis_index("s")
        w = c * NS + t

        @pl.loop(0, zb, step=16)
        def _(j):
            zbuf[pl.ds(j, 16)] = jnp.zeros((16,), jnp.float32)

        @pl.loop(0, tpr, step=zb)
        def _(q):
            pltpu.sync_copy(zbuf, acc.at[pl.ds(t * tpr + q, zb)])

        @pl.loop(0, CB, step=16)
        def _(j):
            ones_v[pl.ds(j, 16)] = jnp.full((16,), 1.0, jnp.float32)

        pltpu.sync_copy(dst_hbm.at[pl.ds(w * rpt, rpt)], didx)
        plsc.subcore_barrier()

        @pl.loop(0, rpt)
        def _(k):
            pltpu.sync_copy(ones_v, acc.at[didx.at[k]], add=True)

        plsc.subcore_barrier()
        pltpu.sync_copy(
            acc.at[pl.ds(t * tpr, tpr)],
            out_hbm.at[pl.ds(c * npad + t * tpr, tpr)],
        )

    return deg_kernel


SCK = 32
NRING = 3


def _make_agg_kernel(nsh, npad, tpr, rpt):
    spc = nsh // NC
    nsup = rpt // SCK

    @functools.partial(
        pl.kernel,
        out_type=jax.ShapeDtypeStruct((nsh, npad, 128), jnp.float32),
        mesh=_MESH,
        scratch_types=[
            pltpu.VMEM((SCK, CB), jnp.int32),
            pltpu.VMEM((SCK, CB), jnp.int32),
            [pltpu.VMEM((CB, 128), jnp.float32)] * NRING,
            pltpu.VMEM_SHARED((npad, 128), jnp.float32),
            [pltpu.SemaphoreType.DMA] * NRING,
            [pltpu.SemaphoreType.DMA] * NRING,
        ],
    )
    def agg_kernel(y_hbm, src_hbm, dst_hbm, out_hbm,
                   srcb, dstb, rbufs, acc, gsems, ssems):
        c = lax.axis_index("c")
        t = lax.axis_index("s")

        for p in range(spc):
            sh = c * spc + p

            @pl.loop(0, CB)
            def _(r):
                @pl.loop(0, 128, step=16)
                def _(j):
                    rbufs[0][r, pl.ds(j, 16)] = jnp.zeros((16,), jnp.float32)

            @pl.loop(0, tpr, step=CB)
            def _(q):
                pltpu.sync_copy(rbufs[0], acc.at[pl.ds(t * tpr + q, CB)])

            plsc.subcore_barrier()

            @pl.loop(0, nsup)
            def _(u):
                base = t * rpt + u * SCK
                pltpu.sync_copy(src_hbm.at[pl.ds(base, SCK)], srcb)
                pltpu.sync_copy(dst_hbm.at[pl.ds(base, SCK)], dstb)
                gds = [None] * SCK
                sds = [None] * SCK
                ytab = y_hbm.at[sh]
                gds[0] = pltpu.async_copy(ytab.at[srcb.at[0]], rbufs[0],
                                          gsems[0])
                for k in range(SCK):
                    b = k % NRING
                    if k + 1 < SCK:
                        nb = (k + 1) % NRING
                        if k + 1 >= NRING:
                            sds[k + 1 - NRING].wait()
                        gds[k + 1] = pltpu.async_copy(
                            ytab.at[srcb.at[k + 1]], rbufs[nb], gsems[nb]
                        )
                    gds[k].wait()
                    sds[k] = pltpu.async_copy(
                        rbufs[b], acc.at[dstb.at[k]], ssems[b], add=True
                    )
                for k in range(SCK - NRING, SCK):
                    sds[k].wait()

            plsc.subcore_barrier()
            pltpu.sync_copy(
                acc.at[pl.ds(t * tpr, tpr)], out_hbm.at[sh, pl.ds(t * tpr, tpr)]
            )

    return agg_kernel


def _dinv_of(dcol_block):
    deg = dcol_block[:, 0:1] + dcol_block[:, 1:2] + 1.0
    return lax.rsqrt(deg)


def _tc_scale(deg_col, x, bm):
    n, d = x.shape

    def body(dcol_ref, x_ref, y_ref):
        y = x_ref[...] * _dinv_of(dcol_ref[...])
        y_ref[0] = y[:, 0:128]
        y_ref[1] = y[:, 128:256]

    return pl.pallas_call(
        body,
        grid=(n // bm,),
        in_specs=[
            pl.BlockSpec((bm, 2), lambda i: (i, 0)),
            pl.BlockSpec((bm, d), lambda i: (i, 0)),
        ],
        out_specs=pl.BlockSpec((2, bm, 128), lambda i: (0, i, 0)),
        out_shape=jax.ShapeDtypeStruct((2, n, 128), jnp.float32),
    )(deg_col, x)


def _tc_layer1(deg_col, y, agg1, W1, b1, bm):
    n = y.shape[1]
    d = 2 * y.shape[2]
    h0 = W1.shape[1]

    def body(dcol_ref, y_ref, a_ref, w_ref, b_ref, o_ref):
        dinv = _dinv_of(dcol_ref[...])
        agg = jnp.concatenate([a_ref[0], a_ref[1]], axis=1)
        yb = jnp.concatenate([y_ref[0], y_ref[1]], axis=1)
        xa = (agg + yb) * dinv
        h = jnp.dot(xa.astype(jnp.bfloat16), w_ref[...],
                    preferred_element_type=jnp.float32)
        h = jnp.maximum(h + b_ref[...], 0.0) * dinv
        for s in range(4):
            o_ref[s] = h[:, 128 * s:128 * (s + 1)]

    return pl.pallas_call(
        body,
        grid=(n // bm,),
        in_specs=[
            pl.BlockSpec((bm, 2), lambda i: (i, 0)),
            pl.BlockSpec((2, bm, 128), lambda i: (0, i, 0)),
            pl.BlockSpec((2, bm, 128), lambda i: (0, i, 0)),
            pl.BlockSpec((d, h0), lambda i: (0, 0)),
            pl.BlockSpec((1, h0), lambda i: (0, 0)),
        ],
        out_specs=pl.BlockSpec((4, bm, 128), lambda i: (0, i, 0)),
        out_shape=jax.ShapeDtypeStruct((4, n, 128), jnp.float32),
    )(deg_col, y, agg1, W1, b1)


def _tc_layer2(deg_col, hs, agg2, Wmu, bmu, Wls, bls, bm):
    n = hs.shape[1]
    h0 = 4 * hs.shape[2]
    h1 = Wmu.shape[1]

    def body(dcol_ref, h_ref, a_ref, wm_ref, bm_ref, wl_ref, bl_ref, o_ref):
        dinv = _dinv_of(dcol_ref[...])
        agg = jnp.concatenate(
            [a_ref[0], a_ref[1], a_ref[2], a_ref[3]], axis=1
        )
        hb = jnp.concatenate(
            [h_ref[0], h_ref[1], h_ref[2], h_ref[3]], axis=1
        )
        ha = ((agg + hb) * dinv).astype(jnp.bfloat16)
        mu = jnp.dot(ha, wm_ref[...], preferred_element_type=jnp.float32)
        mu = mu + bm_ref[...]
        ls = jnp.dot(ha, wl_ref[...], preferred_element_type=jnp.float32)
        ls = ls + bl_ref[...]
        sg = jnp.where(ls > 0.0, ls, jnp.exp(jnp.minimum(ls, 0.0)) - 1.0)
        o_ref[0] = mu
        o_ref[1] = sg + (1.0 + 1e-14)

    return pl.pallas_call(
        body,
        grid=(n // bm,),
        in_specs=[
            pl.BlockSpec((bm, 2), lambda i: (i, 0)),
            pl.BlockSpec((4, bm, 128), lambda i: (0, i, 0)),
            pl.BlockSpec((4, bm, 128), lambda i: (0, i, 0)),
            pl.BlockSpec((h0, h1), lambda i: (0, 0)),
            pl.BlockSpec((1, h1), lambda i: (0, 0)),
            pl.BlockSpec((h0, h1), lambda i: (0, 0)),
            pl.BlockSpec((1, h1), lambda i: (0, 0)),
        ],
        out_specs=pl.BlockSpec((2, bm, h1), lambda i: (0, i, 0)),
        out_shape=jax.ShapeDtypeStruct((2, n, h1), jnp.float32),
    )(deg_col, hs, agg2, Wmu, bmu, Wls, bls)


def kernel(x, edge_index, W1, b1, Wmu, bmu, Wls, bls):
    n, d_in = x.shape
    h0 = W1.shape[1]
    h1 = Wmu.shape[1]
    e = edge_index.shape[1]
    assert d_in % 128 == 0 and h0 % 128 == 0

    tpr = _round_up(-(-n // NS), 160)
    npad = tpr * NS
    sentinel = npad - 1

    src = edge_index[0]
    dst = edge_index[1]

    ea = _round_up(e, NC * NS * CB * 8)
    src_a = _pad_edges(src, ea, 0, n).reshape(ea // CB, CB)
    dst_a = _pad_edges(dst, ea, n, npad - n).reshape(ea // CB, CB)
    rpt_a = ea // CB // NS
    rpt_deg = ea // CB // (NC * NS)

    deg_parts = _make_deg_kernel(npad, tpr, rpt_deg)(dst_a)
    deg_col = jnp.transpose(deg_parts.reshape(NC, npad))

    bm = 1000 if n % 1000 == 0 else n
    y = _tc_scale(deg_col, x, bm)

    agg1 = _make_agg_kernel(2, npad, tpr, rpt_a)(y, src_a, dst_a)
    hs = _tc_layer1(deg_col, y, agg1, W1.astype(jnp.bfloat16),
                    b1.reshape(1, h0), bm)

    agg2 = _make_agg_kernel(4, npad, tpr, rpt_a)(hs, src_a, dst_a)
    return _tc_layer2(
        deg_col, hs, agg2, Wmu.astype(jnp.bfloat16), bmu.reshape(1, h1),
        Wls.astype(jnp.bfloat16), bls.reshape(1, h1), bm
    )


# speedup vs baseline: 1.0842x; 1.0096x over previous
import functools

import jax
import jax.numpy as jnp
from jax import lax
from jax.experimental import pallas as pl
from jax.experimental.pallas import tpu as pltpu
from jax.experimental.pallas import tpu_sc as plsc

NC = 2
NS = 16
CB = 80

_MESH = plsc.VectorSubcoreMesh(
    core_axis_name="c", subcore_axis_name="s", num_cores=NC, num_subcores=NS
)


def _round_up(a, b):
    return (a + b - 1) // b * b


def _pad_edges(v, total, base, spread):
    if v.shape[0] == total:
        return v
    pad = base + jnp.arange(total - v.shape[0], dtype=v.dtype) % spread
    return jnp.concatenate([v, pad])


def _make_deg_kernel(npad, tpr, rpt):
    zb = 160

    @functools.partial(
        pl.kernel,
        out_type=jax.ShapeDtypeStruct((NC * npad,), jnp.float32),
        mesh=_MESH,
        scratch_types=[
            pltpu.VMEM((rpt, CB), jnp.int32),
            pltpu.VMEM((CB,), jnp.float32),
            pltpu.VMEM((zb,), jnp.float32),
            pltpu.VMEM_SHARED((npad,), jnp.float32),
            pltpu.SemaphoreType.DMA,
        ],
    )
    def deg_kernel(dst_hbm, out_hbm, didx, ones_v, zbuf, acc, sem):
        c = lax.axis_index("c")
        t = lax.axis_index("s")
        w = c * NS + t

        @pl.loop(0, zb, step=16)
        def _(j):
            zbuf[pl.ds(j, 16)] = jnp.zeros((16,), jnp.float32)

        @pl.loop(0, tpr, step=zb)
        def _(q):
            pltpu.sync_copy(zbuf, acc.at[pl.ds(t * tpr + q, zb)])

        @pl.loop(0, CB, step=16)
        def _(j):
            ones_v[pl.ds(j, 16)] = jnp.full((16,), 1.0, jnp.float32)

        pltpu.sync_copy(dst_hbm.at[pl.ds(w * rpt, rpt)], didx)
        plsc.subcore_barrier()

        @pl.loop(0, rpt)
        def _(k):
            pltpu.sync_copy(ones_v, acc.at[didx.at[k]], add=True)

        plsc.subcore_barrier()
        pltpu.sync_copy(
            acc.at[pl.ds(t * tpr, tpr)],
            out_hbm.at[pl.ds(c * npad + t * tpr, tpr)],
        )

    return deg_kernel


SCK = 32
NRING = 3


def _make_agg_kernel(nsh, npad, tpr, rpt):
    spc = nsh // NC
    nsup = rpt // SCK

    @functools.partial(
        pl.kernel,
        out_type=jax.ShapeDtypeStruct((nsh, npad, 128), jnp.float32),
        mesh=_MESH,
        scratch_types=[
            pltpu.VMEM((SCK, CB), jnp.int32),
            pltpu.VMEM((SCK, CB), jnp.int32),
            [pltpu.VMEM((CB, 128), jnp.float32)] * NRING,
            pltpu.VMEM_SHARED((npad, 128), jnp.float32),
            [pltpu.SemaphoreType.DMA] * NRING,
            [pltpu.SemaphoreType.DMA] * NRING,
        ],
    )
    def agg_kernel(y_hbm, src_hbm, dst_hbm, out_hbm,
                   srcb, dstb, rbufs, acc, gsems, ssems):
        c = lax.axis_index("c")
        t = lax.axis_index("s")

        for p in range(spc):
            sh = c * spc + p

            @pl.loop(0, CB)
            def _(r):
                @pl.loop(0, 128, step=16)
                def _(j):
                    rbufs[0][r, pl.ds(j, 16)] = jnp.zeros((16,), jnp.float32)

            @pl.loop(0, tpr, step=CB)
            def _(q):
                pltpu.sync_copy(rbufs[0], acc.at[pl.ds(t * tpr + q, CB)])

            plsc.subcore_barrier()

            @pl.loop(0, nsup)
            def _(u):
                base = t * rpt + u * SCK
                pltpu.sync_copy(src_hbm.at[pl.ds(base, SCK)], srcb)
                pltpu.sync_copy(dst_hbm.at[pl.ds(base, SCK)], dstb)
                gds = [None] * SCK
                sds = [None] * SCK
                ytab = y_hbm.at[sh]
                gds[0] = pltpu.async_copy(ytab.at[srcb.at[0]], rbufs[0],
                                          gsems[0])
                for k in range(SCK):
                    b = k % NRING
                    if k + 1 < SCK:
                        nb = (k + 1) % NRING
                        if k + 1 >= NRING:
                            sds[k + 1 - NRING].wait()
                        gds[k + 1] = pltpu.async_copy(
                            ytab.at[srcb.at[k + 1]], rbufs[nb], gsems[nb]
                        )
                    gds[k].wait()
                    sds[k] = pltpu.async_copy(
                        rbufs[b], acc.at[dstb.at[k]], ssems[b], add=True
                    )
                for k in range(SCK - NRING, SCK):
                    sds[k].wait()

            plsc.subcore_barrier()
            pltpu.sync_copy(
                acc.at[pl.ds(t * tpr, tpr)], out_hbm.at[sh, pl.ds(t * tpr, tpr)]
            )

    return agg_kernel


def _dinv_of(dcol_block):
    deg = dcol_block[:, 0:1] + dcol_block[:, 1:2] + 1.0
    return lax.rsqrt(deg)


def _tc_scale(deg_col, x, bm):
    n, d = x.shape

    def body(dcol_ref, x_ref, y_ref):
        y = x_ref[...] * _dinv_of(dcol_ref[...])
        y_ref[0] = y[:, 0:128]
        y_ref[1] = y[:, 128:256]

    return pl.pallas_call(
        body,
        grid=(n // bm,),
        in_specs=[
            pl.BlockSpec((bm, 2), lambda i: (i, 0)),
            pl.BlockSpec((bm, d), lambda i: (i, 0)),
        ],
        out_specs=pl.BlockSpec((2, bm, 128), lambda i: (0, i, 0)),
        out_shape=jax.ShapeDtypeStruct((2, n, 128), jnp.float32),
    )(deg_col, x)


def _tc_layer1(deg_col, y, agg1, W1, b1, bm):
    n = y.shape[1]
    d = 2 * y.shape[2]
    h0 = W1.shape[1]

    def body(dcol_ref, y_ref, a_ref, w_ref, b_ref, o_ref):
        dinv = _dinv_of(dcol_ref[...])
        agg = jnp.concatenate([a_ref[0], a_ref[1]], axis=1)
        yb = jnp.concatenate([y_ref[0], y_ref[1]], axis=1)
        xa = (agg + yb) * dinv
        h = jnp.dot(xa.astype(jnp.bfloat16), w_ref[...],
                    preferred_element_type=jnp.float32)
        h = jnp.maximum(h + b_ref[...], 0.0) * dinv
        for s in range(4):
            o_ref[s] = h[:, 128 * s:128 * (s + 1)]

    return pl.pallas_call(
        body,
        grid=(n // bm,),
        in_specs=[
            pl.BlockSpec((bm, 2), lambda i: (i, 0)),
            pl.BlockSpec((2, bm, 128), lambda i: (0, i, 0)),
            pl.BlockSpec((2, bm, 128), lambda i: (0, i, 0)),
            pl.BlockSpec((d, h0), lambda i: (0, 0)),
            pl.BlockSpec((1, h0), lambda i: (0, 0)),
        ],
        out_specs=pl.BlockSpec((4, bm, 128), lambda i: (0, i, 0)),
        out_shape=jax.ShapeDtypeStruct((4, n, 128), jnp.float32),
    )(deg_col, y, agg1, W1, b1)


def _tc_layer2(deg_col, hs, agg2, Wmu, bmu, Wls, bls, bm):
    n = hs.shape[1]
    h0 = 4 * hs.shape[2]
    h1 = Wmu.shape[1]

    def body(dcol_ref, h_ref, a_ref, wm_ref, bm_ref, wl_ref, bl_ref, o_ref):
        dinv = _dinv_of(dcol_ref[...])
        agg = jnp.concatenate(
            [a_ref[0], a_ref[1], a_ref[2], a_ref[3]], axis=1
        )
        hb = jnp.concatenate(
            [h_ref[0], h_ref[1], h_ref[2], h_ref[3]], axis=1
        )
        ha = ((agg + hb) * dinv).astype(jnp.bfloat16)
        mu = jnp.dot(ha, wm_ref[...], preferred_element_type=jnp.float32)
        mu = mu + bm_ref[...]
        ls = jnp.dot(ha, wl_ref[...], preferred_element_type=jnp.float32)
        ls = ls + bl_ref[...]
        sg = jnp.where(ls > 0.0, ls, jnp.exp(jnp.minimum(ls, 0.0)) - 1.0)
        o_ref[0] = mu
        o_ref[1] = sg + (1.0 + 1e-14)

    return pl.pallas_call(
        body,
        grid=(n // bm,),
        in_specs=[
            pl.BlockSpec((bm, 2), lambda i: (i, 0)),
            pl.BlockSpec((4, bm, 128), lambda i: (0, i, 0)),
            pl.BlockSpec((4, bm, 128), lambda i: (0, i, 0)),
            pl.BlockSpec((h0, h1), lambda i: (0, 0)),
            pl.BlockSpec((1, h1), lambda i: (0, 0)),
            pl.BlockSpec((h0, h1), lambda i: (0, 0)),
            pl.BlockSpec((1, h1), lambda i: (0, 0)),
        ],
        out_specs=pl.BlockSpec((2, bm, h1), lambda i: (0, i, 0)),
        out_shape=jax.ShapeDtypeStruct((2, n, h1), jnp.float32),
    )(deg_col, hs, agg2, Wmu, bmu, Wls, bls)


def kernel(x, edge_index, W1, b1, Wmu, bmu, Wls, bls):
    n, d_in = x.shape
    h0 = W1.shape[1]
    h1 = Wmu.shape[1]
    e = edge_index.shape[1]
    assert d_in % 128 == 0 and h0 % 128 == 0

    tpr = _round_up(-(-n // NS), 160)
    npad = tpr * NS
    sentinel = npad - 1

    src = edge_index[0]
    dst = edge_index[1]

    ea = _round_up(e, NC * NS * CB * 8)
    src_a = _pad_edges(src, ea, 0, n).reshape(ea // CB, CB)
    dst_a = _pad_edges(dst, ea, n, npad - n).reshape(ea // CB, CB)
    rpt_a = ea // CB // NS
    rpt_deg = ea // CB // (NC * NS)

    deg_parts = _make_deg_kernel(npad, tpr, rpt_deg)(dst_a)
    deg_col = jnp.transpose(deg_parts.reshape(NC, npad))

    bm = 2000 if n % 2000 == 0 else n
    y = _tc_scale(deg_col, x, bm)

    agg1 = _make_agg_kernel(2, npad, tpr, rpt_a)(y, src_a, dst_a)
    hs = _tc_layer1(deg_col, y, agg1, W1.astype(jnp.bfloat16),
                    b1.reshape(1, h0), bm)

    agg2 = _make_agg_kernel(4, npad, tpr, rpt_a)(hs, src_a, dst_a)
    return _tc_layer2(
        deg_col, hs, agg2, Wmu.astype(jnp.bfloat16), bmu.reshape(1, h1),
        Wls.astype(jnp.bfloat16), bls.reshape(1, h1), bm
    )


# speedup vs baseline: 1.1245x; 1.0371x over previous
import functools

import jax
import jax.numpy as jnp
from jax import lax
from jax.experimental import pallas as pl
from jax.experimental.pallas import tpu as pltpu
from jax.experimental.pallas import tpu_sc as plsc

NC = 2
NS = 16
CB = 80

_MESH = plsc.VectorSubcoreMesh(
    core_axis_name="c", subcore_axis_name="s", num_cores=NC, num_subcores=NS
)


def _round_up(a, b):
    return (a + b - 1) // b * b


def _pad_edges(v, total, base, spread):
    if v.shape[0] == total:
        return v
    pad = base + jnp.arange(total - v.shape[0], dtype=v.dtype) % spread
    return jnp.concatenate([v, pad])


def _make_deg_kernel(npad, tpr, rpt):
    zb = 160

    @functools.partial(
        pl.kernel,
        out_type=jax.ShapeDtypeStruct((NC * npad,), jnp.float32),
        mesh=_MESH,
        scratch_types=[
            pltpu.VMEM((rpt, CB), jnp.int32),
            pltpu.VMEM((CB,), jnp.float32),
            pltpu.VMEM((zb,), jnp.float32),
            pltpu.VMEM_SHARED((npad,), jnp.float32),
            pltpu.SemaphoreType.DMA,
        ],
    )
    def deg_kernel(dst_hbm, out_hbm, didx, ones_v, zbuf, acc, sem):
        c = lax.axis_index("c")
        t = lax.axis_index("s")
        w = c * NS + t

        @pl.loop(0, zb, step=16)
        def _(j):
            zbuf[pl.ds(j, 16)] = jnp.zeros((16,), jnp.float32)

        @pl.loop(0, tpr, step=zb)
        def _(q):
            pltpu.sync_copy(zbuf, acc.at[pl.ds(t * tpr + q, zb)])

        @pl.loop(0, CB, step=16)
        def _(j):
            ones_v[pl.ds(j, 16)] = jnp.full((16,), 1.0, jnp.float32)

        pltpu.sync_copy(dst_hbm.at[pl.ds(w * rpt, rpt)], didx)
        plsc.subcore_barrier()

        @pl.loop(0, rpt)
        def _(k):
            pltpu.sync_copy(ones_v, acc.at[didx.at[k]], add=True)

        plsc.subcore_barrier()
        pltpu.sync_copy(
            acc.at[pl.ds(t * tpr, tpr)],
            out_hbm.at[pl.ds(c * npad + t * tpr, tpr)],
        )

    return deg_kernel


SCK = 64
NRING = 3


def _make_agg_kernel(nsh, npad, tpr, rpt):
    spc = nsh // NC
    nsup = rpt // SCK

    @functools.partial(
        pl.kernel,
        out_type=jax.ShapeDtypeStruct((nsh, npad, 128), jnp.float32),
        mesh=_MESH,
        scratch_types=[
            pltpu.VMEM((SCK, CB), jnp.int32),
            pltpu.VMEM((SCK, CB), jnp.int32),
            [pltpu.VMEM((CB, 128), jnp.float32)] * NRING,
            pltpu.VMEM_SHARED((npad, 128), jnp.float32),
            [pltpu.SemaphoreType.DMA] * NRING,
            [pltpu.SemaphoreType.DMA] * NRING,
        ],
    )
    def agg_kernel(y_hbm, src_hbm, dst_hbm, out_hbm,
                   srcb, dstb, rbufs, acc, gsems, ssems):
        c = lax.axis_index("c")
        t = lax.axis_index("s")

        for p in range(spc):
            sh = c * spc + p

            @pl.loop(0, CB)
            def _(r):
                @pl.loop(0, 128, step=16)
                def _(j):
                    rbufs[0][r, pl.ds(j, 16)] = jnp.zeros((16,), jnp.float32)

            @pl.loop(0, tpr, step=CB)
            def _(q):
                pltpu.sync_copy(rbufs[0], acc.at[pl.ds(t * tpr + q, CB)])

            plsc.subcore_barrier()

            @pl.loop(0, nsup)
            def _(u):
                base = t * rpt + u * SCK
                pltpu.sync_copy(src_hbm.at[pl.ds(base, SCK)], srcb)
                pltpu.sync_copy(dst_hbm.at[pl.ds(base, SCK)], dstb)
                gds = [None] * SCK
                sds = [None] * SCK
                ytab = y_hbm.at[sh]
                gds[0] = pltpu.async_copy(ytab.at[srcb.at[0]], rbufs[0],
                                          gsems[0])
                for k in range(SCK):
                    b = k % NRING
                    if k + 1 < SCK:
                        nb = (k + 1) % NRING
                        if k + 1 >= NRING:
                            sds[k + 1 - NRING].wait()
                        gds[k + 1] = pltpu.async_copy(
                            ytab.at[srcb.at[k + 1]], rbufs[nb], gsems[nb]
                        )
                    gds[k].wait()
                    sds[k] = pltpu.async_copy(
                        rbufs[b], acc.at[dstb.at[k]], ssems[b], add=True
                    )
                for k in range(SCK - NRING, SCK):
                    sds[k].wait()

            plsc.subcore_barrier()
            pltpu.sync_copy(
                acc.at[pl.ds(t * tpr, tpr)], out_hbm.at[sh, pl.ds(t * tpr, tpr)]
            )

    return agg_kernel


def _dinv_of(dcol_block):
    deg = dcol_block[:, 0:1] + dcol_block[:, 1:2] + 1.0
    return lax.rsqrt(deg)


def _tc_scale(deg_col, x, bm):
    n, d = x.shape

    def body(dcol_ref, x_ref, y_ref):
        y = x_ref[...] * _dinv_of(dcol_ref[...])
        y_ref[0] = y[:, 0:128]
        y_ref[1] = y[:, 128:256]

    return pl.pallas_call(
        body,
        grid=(n // bm,),
        in_specs=[
            pl.BlockSpec((bm, 2), lambda i: (i, 0)),
            pl.BlockSpec((bm, d), lambda i: (i, 0)),
        ],
        out_specs=pl.BlockSpec((2, bm, 128), lambda i: (0, i, 0)),
        out_shape=jax.ShapeDtypeStruct((2, n, 128), jnp.float32),
    )(deg_col, x)


def _tc_layer1(deg_col, y, agg1, W1, b1, bm):
    n = y.shape[1]
    d = 2 * y.shape[2]
    h0 = W1.shape[1]

    def body(dcol_ref, y_ref, a_ref, w_ref, b_ref, o_ref):
        dinv = _dinv_of(dcol_ref[...])
        agg = jnp.concatenate([a_ref[0], a_ref[1]], axis=1)
        yb = jnp.concatenate([y_ref[0], y_ref[1]], axis=1)
        xa = (agg + yb) * dinv
        h = jnp.dot(xa.astype(jnp.bfloat16), w_ref[...],
                    preferred_element_type=jnp.float32)
        h = jnp.maximum(h + b_ref[...], 0.0) * dinv
        for s in range(4):
            o_ref[s] = h[:, 128 * s:128 * (s + 1)]

    return pl.pallas_call(
        body,
        grid=(n // bm,),
        in_specs=[
            pl.BlockSpec((bm, 2), lambda i: (i, 0)),
            pl.BlockSpec((2, bm, 128), lambda i: (0, i, 0)),
            pl.BlockSpec((2, bm, 128), lambda i: (0, i, 0)),
            pl.BlockSpec((d, h0), lambda i: (0, 0)),
            pl.BlockSpec((1, h0), lambda i: (0, 0)),
        ],
        out_specs=pl.BlockSpec((4, bm, 128), lambda i: (0, i, 0)),
        out_shape=jax.ShapeDtypeStruct((4, n, 128), jnp.float32),
    )(deg_col, y, agg1, W1, b1)


def _tc_layer2(deg_col, hs, agg2, Wmu, bmu, Wls, bls, bm):
    n = hs.shape[1]
    h0 = 4 * hs.shape[2]
    h1 = Wmu.shape[1]

    def body(dcol_ref, h_ref, a_ref, wm_ref, bm_ref, wl_ref, bl_ref, o_ref):
        dinv = _dinv_of(dcol_ref[...])
        agg = jnp.concatenate(
            [a_ref[0], a_ref[1], a_ref[2], a_ref[3]], axis=1
        )
        hb = jnp.concatenate(
            [h_ref[0], h_ref[1], h_ref[2], h_ref[3]], axis=1
        )
        ha = ((agg + hb) * dinv).astype(jnp.bfloat16)
        mu = jnp.dot(ha, wm_ref[...], preferred_element_type=jnp.float32)
        mu = mu + bm_ref[...]
        ls = jnp.dot(ha, wl_ref[...], preferred_element_type=jnp.float32)
        ls = ls + bl_ref[...]
        sg = jnp.where(ls > 0.0, ls, jnp.exp(jnp.minimum(ls, 0.0)) - 1.0)
        o_ref[0] = mu
        o_ref[1] = sg + (1.0 + 1e-14)

    return pl.pallas_call(
        body,
        grid=(n // bm,),
        in_specs=[
            pl.BlockSpec((bm, 2), lambda i: (i, 0)),
            pl.BlockSpec((4, bm, 128), lambda i: (0, i, 0)),
            pl.BlockSpec((4, bm, 128), lambda i: (0, i, 0)),
            pl.BlockSpec((h0, h1), lambda i: (0, 0)),
            pl.BlockSpec((1, h1), lambda i: (0, 0)),
            pl.BlockSpec((h0, h1), lambda i: (0, 0)),
            pl.BlockSpec((1, h1), lambda i: (0, 0)),
        ],
        out_specs=pl.BlockSpec((2, bm, h1), lambda i: (0, i, 0)),
        out_shape=jax.ShapeDtypeStruct((2, n, h1), jnp.float32),
    )(deg_col, hs, agg2, Wmu, bmu, Wls, bls)


def kernel(x, edge_index, W1, b1, Wmu, bmu, Wls, bls):
    n, d_in = x.shape
    h0 = W1.shape[1]
    h1 = Wmu.shape[1]
    e = edge_index.shape[1]
    assert d_in % 128 == 0 and h0 % 128 == 0

    tpr = _round_up(-(-n // NS), 160)
    npad = tpr * NS
    sentinel = npad - 1

    src = edge_index[0]
    dst = edge_index[1]

    ea = _round_up(e, NC * NS * CB * 8)
    src_a = _pad_edges(src, ea, 0, n).reshape(ea // CB, CB)
    dst_a = _pad_edges(dst, ea, n, npad - n).reshape(ea // CB, CB)
    rpt_a = ea // CB // NS
    rpt_deg = ea // CB // (NC * NS)

    deg_parts = _make_deg_kernel(npad, tpr, rpt_deg)(dst_a)
    deg_col = jnp.transpose(deg_parts.reshape(NC, npad))

    bm = 2000 if n % 2000 == 0 else n
    y = _tc_scale(deg_col, x, bm)

    agg1 = _make_agg_kernel(2, npad, tpr, rpt_a)(y, src_a, dst_a)
    hs = _tc_layer1(deg_col, y, agg1, W1.astype(jnp.bfloat16),
                    b1.reshape(1, h0), bm)

    agg2 = _make_agg_kernel(4, npad, tpr, rpt_a)(hs, src_a, dst_a)
    return _tc_layer2(
        deg_col, hs, agg2, Wmu.astype(jnp.bfloat16), bmu.reshape(1, h1),
        Wls.astype(jnp.bfloat16), bls.reshape(1, h1), bm
    )


# speedup vs baseline: 1.1258x; 1.0011x over previous
import functools

import jax
import jax.numpy as jnp
from jax import lax
from jax.experimental import pallas as pl
from jax.experimental.pallas import tpu as pltpu
from jax.experimental.pallas import tpu_sc as plsc

NC = 2
NS = 16
CB = 80

_MESH = plsc.VectorSubcoreMesh(
    core_axis_name="c", subcore_axis_name="s", num_cores=NC, num_subcores=NS
)


def _round_up(a, b):
    return (a + b - 1) // b * b


def _pad_edges(v, total, base, spread):
    if v.shape[0] == total:
        return v
    pad = base + jnp.arange(total - v.shape[0], dtype=v.dtype) % spread
    return jnp.concatenate([v, pad])


def _make_deg_kernel(npad, tpr, rpt):
    zb = 160

    @functools.partial(
        pl.kernel,
        out_type=jax.ShapeDtypeStruct((NC * npad,), jnp.float32),
        mesh=_MESH,
        scratch_types=[
            pltpu.VMEM((rpt, CB), jnp.int32),
            pltpu.VMEM((CB,), jnp.float32),
            pltpu.VMEM((zb,), jnp.float32),
            pltpu.VMEM_SHARED((npad,), jnp.float32),
            pltpu.SemaphoreType.DMA,
        ],
    )
    def deg_kernel(dst_hbm, out_hbm, didx, ones_v, zbuf, acc, sem):
        c = lax.axis_index("c")
        t = lax.axis_index("s")
        w = c * NS + t

        @pl.loop(0, zb, step=16)
        def _(j):
            zbuf[pl.ds(j, 16)] = jnp.zeros((16,), jnp.float32)

        @pl.loop(0, tpr, step=zb)
        def _(q):
            pltpu.sync_copy(zbuf, acc.at[pl.ds(t * tpr + q, zb)])

        @pl.loop(0, CB, step=16)
        def _(j):
            ones_v[pl.ds(j, 16)] = jnp.full((16,), 1.0, jnp.float32)

        pltpu.sync_copy(dst_hbm.at[pl.ds(w * rpt, rpt)], didx)
        plsc.subcore_barrier()

        @pl.loop(0, rpt)
        def _(k):
            pltpu.sync_copy(ones_v, acc.at[didx.at[k]], add=True)

        plsc.subcore_barrier()
        pltpu.sync_copy(
            acc.at[pl.ds(t * tpr, tpr)],
            out_hbm.at[pl.ds(c * npad + t * tpr, tpr)],
        )

    return deg_kernel


SCK = 64
NRING = 3


def _make_agg_kernel(nsh, npad, tpr, rpt):
    spc = nsh // NC
    nsup = rpt // SCK

    @functools.partial(
        pl.kernel,
        out_type=jax.ShapeDtypeStruct((nsh, npad, 128), jnp.float32),
        mesh=_MESH,
        scratch_types=[
            pltpu.VMEM((SCK, CB), jnp.int32),
            pltpu.VMEM((SCK, CB), jnp.int32),
            [pltpu.VMEM((CB, 128), jnp.float32)] * NRING,
            pltpu.VMEM_SHARED((npad, 128), jnp.float32),
            [pltpu.SemaphoreType.DMA] * NRING,
            [pltpu.SemaphoreType.DMA] * NRING,
        ],
    )
    def agg_kernel(y_hbm, src_hbm, dst_hbm, out_hbm,
                   srcb, dstb, rbufs, acc, gsems, ssems):
        c = lax.axis_index("c")
        t = lax.axis_index("s")

        for p in range(spc):
            sh = c * spc + p

            @pl.loop(0, CB)
            def _(r):
                @pl.loop(0, 128, step=16)
                def _(j):
                    rbufs[0][r, pl.ds(j, 16)] = jnp.zeros((16,), jnp.float32)

            @pl.loop(0, tpr, step=CB)
            def _(q):
                pltpu.sync_copy(rbufs[0], acc.at[pl.ds(t * tpr + q, CB)])

            plsc.subcore_barrier()

            @pl.loop(0, nsup)
            def _(u):
                base = t * rpt + u * SCK
                pltpu.sync_copy(src_hbm.at[pl.ds(base, SCK)], srcb)
                pltpu.sync_copy(dst_hbm.at[pl.ds(base, SCK)], dstb)
                gds = [None] * SCK
                sds = [None] * SCK
                ytab = y_hbm.at[sh]
                gds[0] = pltpu.async_copy(ytab.at[srcb.at[0]], rbufs[0],
                                          gsems[0])
                for k in range(SCK):
                    b = k % NRING
                    if k + 1 < SCK:
                        nb = (k + 1) % NRING
                        if k + 1 >= NRING:
                            sds[k + 1 - NRING].wait()
                        gds[k + 1] = pltpu.async_copy(
                            ytab.at[srcb.at[k + 1]], rbufs[nb], gsems[nb]
                        )
                    gds[k].wait()
                    sds[k] = pltpu.async_copy(
                        rbufs[b], acc.at[dstb.at[k]], ssems[b], add=True
                    )
                for k in range(SCK - NRING, SCK):
                    sds[k].wait()

            plsc.subcore_barrier()
            pltpu.sync_copy(
                acc.at[pl.ds(t * tpr, tpr)], out_hbm.at[sh, pl.ds(t * tpr, tpr)]
            )

    return agg_kernel


def _dinv_of(dcol_block):
    deg = dcol_block[:, 0:1] + dcol_block[:, 1:2] + 1.0
    return lax.rsqrt(deg)


def _tc_scale(deg_col, x, bm):
    n, d = x.shape

    def body(dcol_ref, x_ref, y_ref):
        y = x_ref[...] * _dinv_of(dcol_ref[...])
        y_ref[0] = y[:, 0:128]
        y_ref[1] = y[:, 128:256]

    return pl.pallas_call(
        body,
        grid=(n // bm,),
        in_specs=[
            pl.BlockSpec((bm, 2), lambda i: (i, 0)),
            pl.BlockSpec((bm, d), lambda i: (i, 0)),
        ],
        out_specs=pl.BlockSpec((2, bm, 128), lambda i: (0, i, 0)),
        out_shape=jax.ShapeDtypeStruct((2, n, 128), jnp.float32),
    )(deg_col, x)


def _tc_layer1(deg_col, y, agg1, W1, b1, bm):
    n = y.shape[1]
    d = 2 * y.shape[2]
    h0 = W1.shape[1]

    def body(dcol_ref, y_ref, a_ref, w_ref, b_ref, o_ref):
        dinv = _dinv_of(dcol_ref[...])
        agg = jnp.concatenate([a_ref[0], a_ref[1]], axis=1)
        yb = jnp.concatenate([y_ref[0], y_ref[1]], axis=1)
        xa = (agg + yb) * dinv
        h = jnp.dot(xa.astype(jnp.bfloat16), w_ref[...],
                    preferred_element_type=jnp.float32)
        h = jnp.maximum(h + b_ref[...], 0.0) * dinv
        for s in range(4):
            o_ref[s] = h[:, 128 * s:128 * (s + 1)]

    return pl.pallas_call(
        body,
        grid=(n // bm,),
        in_specs=[
            pl.BlockSpec((bm, 2), lambda i: (i, 0)),
            pl.BlockSpec((2, bm, 128), lambda i: (0, i, 0)),
            pl.BlockSpec((2, bm, 128), lambda i: (0, i, 0)),
            pl.BlockSpec((d, h0), lambda i: (0, 0)),
            pl.BlockSpec((1, h0), lambda i: (0, 0)),
        ],
        out_specs=pl.BlockSpec((4, bm, 128), lambda i: (0, i, 0)),
        out_shape=jax.ShapeDtypeStruct((4, n, 128), jnp.float32),
    )(deg_col, y, agg1, W1, b1)


def _tc_layer2(deg_col, hs, agg2, Wmu, bmu, Wls, bls, bm):
    n = hs.shape[1]
    h0 = 4 * hs.shape[2]
    h1 = Wmu.shape[1]

    def body(dcol_ref, h_ref, a_ref, wm_ref, bm_ref, wl_ref, bl_ref, o_ref):
        dinv = _dinv_of(dcol_ref[...])
        agg = jnp.concatenate(
            [a_ref[0], a_ref[1], a_ref[2], a_ref[3]], axis=1
        )
        hb = jnp.concatenate(
            [h_ref[0], h_ref[1], h_ref[2], h_ref[3]], axis=1
        )
        ha = ((agg + hb) * dinv).astype(jnp.bfloat16)
        mu = jnp.dot(ha, wm_ref[...], preferred_element_type=jnp.float32)
        mu = mu + bm_ref[...]
        ls = jnp.dot(ha, wl_ref[...], preferred_element_type=jnp.float32)
        ls = ls + bl_ref[...]
        sg = jnp.where(ls > 0.0, ls, jnp.exp(jnp.minimum(ls, 0.0)) - 1.0)
        o_ref[0] = mu
        o_ref[1] = sg + (1.0 + 1e-14)

    return pl.pallas_call(
        body,
        grid=(n // bm,),
        in_specs=[
            pl.BlockSpec((bm, 2), lambda i: (i, 0)),
            pl.BlockSpec((4, bm, 128), lambda i: (0, i, 0)),
            pl.BlockSpec((4, bm, 128), lambda i: (0, i, 0)),
            pl.BlockSpec((h0, h1), lambda i: (0, 0)),
            pl.BlockSpec((1, h1), lambda i: (0, 0)),
            pl.BlockSpec((h0, h1), lambda i: (0, 0)),
            pl.BlockSpec((1, h1), lambda i: (0, 0)),
        ],
        out_specs=pl.BlockSpec((2, bm, h1), lambda i: (0, i, 0)),
        out_shape=jax.ShapeDtypeStruct((2, n, h1), jnp.float32),
    )(deg_col, hs, agg2, Wmu, bmu, Wls, bls)


def kernel(x, edge_index, W1, b1, Wmu, bmu, Wls, bls):
    n, d_in = x.shape
    h0 = W1.shape[1]
    h1 = Wmu.shape[1]
    e = edge_index.shape[1]
    assert d_in % 128 == 0 and h0 % 128 == 0

    tpr = _round_up(-(-n // NS), 160)
    npad = tpr * NS
    sentinel = npad - 1

    src = edge_index[0]
    dst = edge_index[1]

    ea = _round_up(e, NC * NS * CB * 8)
    src_a = _pad_edges(src, ea, 0, n).reshape(ea // CB, CB)
    dst_a = _pad_edges(dst, ea, n, npad - n).reshape(ea // CB, CB)
    rpt_a = ea // CB // NS
    rpt_deg = ea // CB // (NC * NS)
    assert rpt_a % SCK == 0 and rpt_deg % 8 == 0 and tpr % CB == 0

    deg_parts = _make_deg_kernel(npad, tpr, rpt_deg)(dst_a)
    deg_col = jnp.transpose(deg_parts.reshape(NC, npad))

    bm = 2000 if n % 2000 == 0 else n
    y = _tc_scale(deg_col, x, bm)

    agg1 = _make_agg_kernel(2, npad, tpr, rpt_a)(y, src_a, dst_a)
    hs = _tc_layer1(deg_col, y, agg1, W1.astype(jnp.bfloat16),
                    b1.reshape(1, h0), bm)

    agg2 = _make_agg_kernel(4, npad, tpr, rpt_a)(hs, src_a, dst_a)
    return _tc_layer2(
        deg_col, hs, agg2, Wmu.astype(jnp.bfloat16), bmu.reshape(1, h1),
        Wls.astype(jnp.bfloat16), bls.reshape(1, h1), bm
    )
